# baseline (speedup 1.0000x reference)
import functools

import jax
import jax.numpy as jnp
from jax import lax
from jax.experimental import pallas as pl
from jax.experimental.pallas import tpu as pltpu
from jax.experimental.pallas import tpu_sc as plsc

VOCAB = 50257
SEQ_LEN = 2048
HIDDEN = 1024
BATCH = 4

NC = 2
NS = 16
LANES = 16
NW = NC * NS

TOKENS = BATCH * SEQ_LEN
TPW = TOKENS // NW
CHUNK = 16
NCHUNK = TPW // CHUNK
VECS = CHUNK * HIDDEN // LANES


def _body(ids_hbm, wte_hbm, wpe_hbm, out_hbm, idx_v, rows_v, pos_v, sem):
    wid = lax.axis_index("s") * NC + lax.axis_index("c")
    base = wid * TPW
    pos_base = lax.rem(base, SEQ_LEN)

    pltpu.sync_copy(ids_hbm.at[pl.ds(base, TPW)], idx_v)

    def chunk_step(c, _):
        tok0 = base + c * CHUNK
        gat = pltpu.async_copy(
            wte_hbm.at[idx_v.at[pl.ds(c * CHUNK, CHUNK)]], rows_v, sem)
        pltpu.sync_copy(wpe_hbm.at[pl.ds(pos_base + c * CHUNK, CHUNK)], pos_v)
        gat.wait()

        def add_step(v, _):
            r = v // (HIDDEN // LANES)
            col = (v % (HIDDEN // LANES)) * LANES
            rows_v[r, pl.ds(col, LANES)] = (
                rows_v[r, pl.ds(col, LANES)] + pos_v[r, pl.ds(col, LANES)])
            return _

        lax.fori_loop(0, VECS, add_step, None)
        pltpu.sync_copy(rows_v, out_hbm.at[pl.ds(tok0, CHUNK)])
        return _

    lax.fori_loop(0, NCHUNK, chunk_step, None)


@jax.jit
def _embed(ids_flat, wte, wpe):
    mesh = plsc.VectorSubcoreMesh(core_axis_name="c", subcore_axis_name="s")
    return pl.kernel(
        _body,
        out_type=jax.ShapeDtypeStruct((TOKENS, HIDDEN), jnp.float32),
        mesh=mesh,
        scratch_types=[
            pltpu.VMEM((TPW,), jnp.int32),
            pltpu.VMEM((CHUNK, HIDDEN), jnp.float32),
            pltpu.VMEM((CHUNK, HIDDEN), jnp.float32),
            pltpu.SemaphoreType.DMA,
        ],
    )(ids_flat, wte, wpe)


def kernel(input_ids, token_embeddings, position_embeddings):
    ids_flat = input_ids.reshape(-1).astype(jnp.int32)
    out = _embed(ids_flat, token_embeddings, position_embeddings)
    return out.reshape(BATCH, SEQ_LEN, HIDDEN)


# speedup vs baseline: 1.1927x; 1.1927x over previous
import jax
import jax.numpy as jnp
from jax import lax
from jax.experimental import pallas as pl
from jax.experimental.pallas import tpu as pltpu
from jax.experimental.pallas import tpu_sc as plsc

VOCAB = 50257
SEQ_LEN = 2048
HIDDEN = 1024
BATCH = 4

NC = 2
NS = 16
LANES = 16
NW = NC * NS

TOKENS = BATCH * SEQ_LEN
POSW = SEQ_LEN // NW
TPW = POSW * BATCH
CHUNK = 16
NCHUNK = TPW // CHUNK
QPB = POSW // CHUNK
VPR = HIDDEN // LANES


def _body(ids_hbm, wte_hbm, wpe_hbm, out_hbm, idx_v, pos_v, rows_v, sem):
    wid = lax.axis_index("s") * NC + lax.axis_index("c")
    p0 = wid * POSW

    pltpu.sync_copy(wpe_hbm.at[pl.ds(p0, POSW)], pos_v)
    for b in range(BATCH):
        pltpu.sync_copy(ids_hbm.at[pl.ds(b * SEQ_LEN + p0, POSW)],
                        idx_v.at[pl.ds(b * POSW, POSW)])

    def chunk_step(c, _):
        b = c // QPB
        q = c % QPB
        tok0 = b * SEQ_LEN + p0 + q * CHUNK
        pltpu.async_copy(
            wte_hbm.at[idx_v.at[pl.ds(c * CHUNK, CHUNK)]], rows_v, sem).wait()

        def row_step(r, _):
            pr = q * CHUNK + r
            for j in range(VPR):
                plsc.addupdate(rows_v.at[r, pl.ds(j * LANES, LANES)],
                               pos_v[pr, pl.ds(j * LANES, LANES)])
            return _

        lax.fori_loop(0, CHUNK, row_step, None)
        pltpu.sync_copy(rows_v, out_hbm.at[pl.ds(tok0, CHUNK)])
        return _

    lax.fori_loop(0, NCHUNK, chunk_step, None)


@jax.jit
def _embed(ids_flat, wte, wpe):
    mesh = plsc.VectorSubcoreMesh(core_axis_name="c", subcore_axis_name="s")
    return pl.kernel(
        _body,
        out_type=jax.ShapeDtypeStruct((TOKENS, HIDDEN), jnp.float32),
        mesh=mesh,
        scratch_types=[
            pltpu.VMEM((TPW,), jnp.int32),
            pltpu.VMEM((POSW, HIDDEN), jnp.float32),
            pltpu.VMEM((CHUNK, HIDDEN), jnp.float32),
            pltpu.SemaphoreType.DMA,
        ],
    )(ids_flat, wte, wpe)


def kernel(input_ids, token_embeddings, position_embeddings):
    ids_flat = input_ids.reshape(-1).astype(jnp.int32)
    out = _embed(ids_flat, token_embeddings, position_embeddings)
    return out.reshape(BATCH, SEQ_LEN, HIDDEN)


# speedup vs baseline: 1.2924x; 1.0836x over previous
import jax
import jax.numpy as jnp
from jax import lax
from jax.experimental import pallas as pl
from jax.experimental.pallas import tpu as pltpu
from jax.experimental.pallas import tpu_sc as plsc

VOCAB = 50257
SEQ_LEN = 2048
HIDDEN = 1024
BATCH = 4

NC = 2
NS = 16
LANES = 16
NW = NC * NS

TOKENS = BATCH * SEQ_LEN
POSW = SEQ_LEN // NW
TPW = POSW * BATCH
CHUNK = 8
NCHUNK = TPW // CHUNK
QPB = POSW // CHUNK
VPR = HIDDEN // LANES
NBUF = 4
NGROUP = NCHUNK // NBUF


def _body(ids_hbm, wte_hbm, wpe_hbm, out_hbm, idx_v, pos_v,
          r0, r1, r2, r3, g0, g1, g2, g3, s0, s1, s2, s3):
    bufs = (r0, r1, r2, r3)
    gsems = (g0, g1, g2, g3)
    ssems = (s0, s1, s2, s3)

    wid = lax.axis_index("s") * NC + lax.axis_index("c")
    p0 = wid * POSW

    pltpu.sync_copy(wpe_hbm.at[pl.ds(p0, POSW)], pos_v)
    for b in range(BATCH):
        pltpu.sync_copy(ids_hbm.at[pl.ds(b * SEQ_LEN + p0, POSW)],
                        idx_v.at[pl.ds(b * POSW, POSW)])

    def g_src(c):
        return wte_hbm.at[idx_v.at[pl.ds(c * CHUNK, CHUNK)]]

    def out_dst(c):
        b = c // QPB
        q = c % QPB
        return out_hbm.at[pl.ds(b * SEQ_LEN + p0 + q * CHUNK, CHUNK)]

    for i in range(NBUF):
        pltpu.async_copy(g_src(i), bufs[i], gsems[i])

    def group(p, _):
        c0 = p * NBUF
        for i in range(NBUF):
            c = c0 + i
            pltpu.make_async_copy(g_src(c), bufs[i], gsems[i]).wait()

            def row_step(r, _, i=i, c=c):
                pr = lax.rem(c, QPB) * CHUNK + r
                for j in range(VPR):
                    plsc.addupdate(bufs[i].at[r, pl.ds(j * LANES, LANES)],
                                   pos_v[pr, pl.ds(j * LANES, LANES)])
                return _

            lax.fori_loop(0, CHUNK, row_step, None)
            pltpu.async_copy(bufs[i], out_dst(c), ssems[i])
        for i in range(NBUF):
            c = c0 + i
            pltpu.make_async_copy(bufs[i], out_dst(c), ssems[i]).wait()

            @pl.when(p < NGROUP - 1)
            def _rearm(i=i, c=c):
                pltpu.async_copy(g_src(c + NBUF), bufs[i], gsems[i])

        return _

    lax.fori_loop(0, NGROUP, group, None)


@jax.jit
def _embed(ids_flat, wte, wpe):
    mesh = plsc.VectorSubcoreMesh(core_axis_name="c", subcore_axis_name="s")
    return pl.kernel(
        _body,
        out_type=jax.ShapeDtypeStruct((TOKENS, HIDDEN), jnp.float32),
        mesh=mesh,
        scratch_types=[
            pltpu.VMEM((TPW,), jnp.int32),
            pltpu.VMEM((POSW, HIDDEN), jnp.float32),
        ] + [pltpu.VMEM((CHUNK, HIDDEN), jnp.float32)] * NBUF
          + [pltpu.SemaphoreType.DMA] * (2 * NBUF),
    )(ids_flat, wte, wpe)


def kernel(input_ids, token_embeddings, position_embeddings):
    ids_flat = input_ids.reshape(-1).astype(jnp.int32)
    out = _embed(ids_flat, token_embeddings, position_embeddings)
    return out.reshape(BATCH, SEQ_LEN, HIDDEN)


# speedup vs baseline: 2.3168x; 1.7926x over previous
import jax
import jax.numpy as jnp
from jax import lax
from jax.experimental import pallas as pl
from jax.experimental.pallas import tpu as pltpu
from jax.experimental.pallas import tpu_sc as plsc

VOCAB = 50257
SEQ_LEN = 2048
HIDDEN = 1024
BATCH = 4

NC = 2
NS = 16
LANES = 16
NW = NC * NS

TOKENS = BATCH * SEQ_LEN
POSW = SEQ_LEN // NW
TPW = POSW * BATCH
CHUNK = 8
NCHUNK = TPW // CHUNK
QPB = POSW // CHUNK
VPR = HIDDEN // LANES
NBUF = 4
NGROUP = NCHUNK // NBUF


def _body(ids_hbm, wte_hbm, wpe_hbm, out_hbm, idx_v, pos_v,
          r0, r1, r2, r3, g0, g1, g2, g3, s0, s1, s2, s3):
    bufs = (r0, r1, r2, r3)
    gsems = (g0, g1, g2, g3)
    ssems = (s0, s1, s2, s3)

    wid = lax.axis_index("s") * NC + lax.axis_index("c")
    p0 = wid * POSW

    pltpu.sync_copy(wpe_hbm.at[pl.ds(p0, POSW)], pos_v)
    for b in range(BATCH):
        pltpu.sync_copy(ids_hbm.at[pl.ds(b * SEQ_LEN + p0, POSW)],
                        idx_v.at[pl.ds(b * POSW, POSW)])

    def g_src(c):
        return wte_hbm.at[idx_v.at[pl.ds(c * CHUNK, CHUNK)]]

    def out_dst(c):
        b = c // QPB
        q = c % QPB
        return out_hbm.at[pl.ds(b * SEQ_LEN + p0 + q * CHUNK, CHUNK)]

    for i in range(NBUF):
        pltpu.async_copy(g_src(i), bufs[i], gsems[i])

    def group(p, _):
        c0 = p * NBUF
        for i in range(NBUF):
            c = c0 + i
            pltpu.make_async_copy(g_src(c), bufs[i], gsems[i]).wait()

            q = lax.rem(c, QPB)

            @plsc.parallel_loop(0, CHUNK * VPR, 1, unroll=8)
            def _add_v(v, i=i, q=q):
                r = v >> 6
                jcol = (v & (VPR - 1)) * LANES
                plsc.addupdate(bufs[i].at[r, pl.ds(jcol, LANES)],
                               pos_v[q * CHUNK + r, pl.ds(jcol, LANES)])
            pltpu.async_copy(bufs[i], out_dst(c), ssems[i])
        for i in range(NBUF):
            c = c0 + i
            pltpu.make_async_copy(bufs[i], out_dst(c), ssems[i]).wait()

            @pl.when(p < NGROUP - 1)
            def _rearm(i=i, c=c):
                pltpu.async_copy(g_src(c + NBUF), bufs[i], gsems[i])

        return _

    lax.fori_loop(0, NGROUP, group, None)


@jax.jit
def _embed(ids_flat, wte, wpe):
    mesh = plsc.VectorSubcoreMesh(core_axis_name="c", subcore_axis_name="s")
    return pl.kernel(
        _body,
        out_type=jax.ShapeDtypeStruct((TOKENS, HIDDEN), jnp.float32),
        mesh=mesh,
        scratch_types=[
            pltpu.VMEM((TPW,), jnp.int32),
            pltpu.VMEM((POSW, HIDDEN), jnp.float32),
        ] + [pltpu.VMEM((CHUNK, HIDDEN), jnp.float32)] * NBUF
          + [pltpu.SemaphoreType.DMA] * (2 * NBUF),
    )(ids_flat, wte, wpe)


def kernel(input_ids, token_embeddings, position_embeddings):
    ids_flat = input_ids.reshape(-1).astype(jnp.int32)
    out = _embed(ids_flat, token_embeddings, position_embeddings)
    return out.reshape(BATCH, SEQ_LEN, HIDDEN)


# speedup vs baseline: 2.3979x; 1.0350x over previous
import jax
import jax.numpy as jnp
from jax import lax
from jax.experimental import pallas as pl
from jax.experimental.pallas import tpu as pltpu
from jax.experimental.pallas import tpu_sc as plsc

VOCAB = 50257
SEQ_LEN = 2048
HIDDEN = 1024
BATCH = 4

NC = 2
NS = 16
LANES = 16
NW = NC * NS

TOKENS = BATCH * SEQ_LEN
POSW = SEQ_LEN // NW
CHUNK = 16
NBLK = POSW // CHUNK
VPR = HIDDEN // LANES
NBUF = BATCH


def _body(ids_hbm, wte_hbm, wpe_hbm, out_hbm, idx_v, pa, pb,
          r0, r1, r2, r3, isem, psem, g0, g1, g2, g3, s0, s1, s2, s3):
    bufs = (r0, r1, r2, r3)
    posb = (pa, pb)
    gsems = (g0, g1, g2, g3)
    ssems = (s0, s1, s2, s3)

    wid = lax.axis_index("s") * NC + lax.axis_index("c")
    p0 = wid * POSW

    def g_src(p, b):
        return wte_hbm.at[idx_v.at[b, pl.ds(p * CHUNK, CHUNK)]]

    def out_dst(p, b):
        return out_hbm.at[pl.ds(b * SEQ_LEN + p0 + p * CHUNK, CHUNK)]

    def pos_src(p):
        return wpe_hbm.at[pl.ds(p0 + p * CHUNK, CHUNK)]

    for b in range(BATCH):
        pltpu.async_copy(ids_hbm.at[pl.ds(b * SEQ_LEN + p0, POSW)],
                         idx_v.at[b], isem)
    pltpu.async_copy(pos_src(0), pa, psem)
    for b in range(BATCH):
        pltpu.make_async_copy(ids_hbm.at[pl.ds(b * SEQ_LEN + p0, POSW)],
                              idx_v.at[b], isem).wait()
    for b in range(NBUF):
        pltpu.async_copy(g_src(0, b), bufs[b], gsems[b])

    for p in range(NBLK):
        pos_v = posb[p % 2]
        pltpu.make_async_copy(pos_src(p), pos_v, psem).wait()
        if p < NBLK - 1:
            pltpu.async_copy(pos_src(p + 1), posb[(p + 1) % 2], psem)
        for b in range(NBUF):
            pltpu.make_async_copy(g_src(p, b), bufs[b], gsems[b]).wait()

            @plsc.parallel_loop(0, CHUNK * VPR, 1, unroll=8)
            def _add_v(v, b=b, pos_v=pos_v):
                r = v >> 6
                jcol = (v & (VPR - 1)) * LANES
                plsc.addupdate(bufs[b].at[r, pl.ds(jcol, LANES)],
                               pos_v[r, pl.ds(jcol, LANES)])

            pltpu.async_copy(bufs[b], out_dst(p, b), ssems[b])
        for b in range(NBUF):
            pltpu.make_async_copy(bufs[b], out_dst(p, b), ssems[b]).wait()
            if p < NBLK - 1:
                pltpu.async_copy(g_src(p + 1, b), bufs[b], gsems[b])


@jax.jit
def _embed(ids, wte, wpe):
    mesh = plsc.VectorSubcoreMesh(core_axis_name="c", subcore_axis_name="s")
    return pl.kernel(
        _body,
        out_type=jax.ShapeDtypeStruct((TOKENS, HIDDEN), jnp.float32),
        mesh=mesh,
        scratch_types=[
            pltpu.VMEM((BATCH, POSW), jnp.int32),
            pltpu.VMEM((CHUNK, HIDDEN), jnp.float32),
            pltpu.VMEM((CHUNK, HIDDEN), jnp.float32),
        ] + [pltpu.VMEM((CHUNK, HIDDEN), jnp.float32)] * NBUF
          + [pltpu.SemaphoreType.DMA] * (2 + 2 * NBUF),
    )(ids, wte, wpe)


def kernel(input_ids, token_embeddings, position_embeddings):
    ids = input_ids.reshape(-1).astype(jnp.int32)
    out = _embed(ids, token_embeddings, position_embeddings)
    return out.reshape(BATCH, SEQ_LEN, HIDDEN)
